# baseline (speedup 1.0000x reference)
import functools

import jax
import jax.numpy as jnp
from jax.experimental import pallas as pl
from jax.experimental.pallas import tpu as pltpu

_VOXEL_SIZE = 0.05

_CompilerParams = getattr(pltpu, "CompilerParams", None) or getattr(
    pltpu, "TPUCompilerParams"
)


def _lin_kernel(pts_ref, lin_ref, *, voxel_size):
    pts = pts_ref[0]
    mn = jnp.min(pts, axis=1, keepdims=True)
    v = jnp.floor((pts - mn) / voxel_size).astype(jnp.int32)
    dims = jnp.max(v, axis=1, keepdims=True) + 1
    d1 = dims[1:2, :]
    d2 = dims[2:3, :]
    lin = (v[0:1, :] * d1 + v[1:2, :]) * d2 + v[2:3, :]
    lin_ref[...] = lin[None]


def _mean_kernel(sums_ref, cnt_ref, out_ref):
    out_ref[...] = sums_ref[...] / jnp.maximum(cnt_ref[...], 1.0)


def _voxel_layer(x, voxel_size, L, key):
    B, C, N = x.shape
    lin = pl.pallas_call(
        functools.partial(_lin_kernel, voxel_size=voxel_size),
        grid=(B,),
        in_specs=[pl.BlockSpec((1, C, N), lambda b: (b, 0, 0))],
        out_specs=pl.BlockSpec((1, 1, N), lambda b: (b, 0, 0)),
        out_shape=jax.ShapeDtypeStruct((B, 1, N), jnp.int32),
        compiler_params=_CompilerParams(dimension_semantics=("parallel",)),
    )(x)[:, 0, :]

    order = jnp.argsort(lin, axis=-1)
    sl = jnp.take_along_axis(lin, order, axis=-1)
    is_new = jnp.concatenate(
        [jnp.ones((B, 1), bool), sl[:, 1:] != sl[:, :-1]], axis=1
    )
    seg = jnp.cumsum(is_new, axis=-1) - 1
    num_vox = seg[:, -1] + 1

    pts_sorted = jnp.take_along_axis(x, order[:, None, :], axis=-1)
    sums = jax.vmap(
        lambda p, s: jax.ops.segment_sum(p.T, s, num_segments=N)
    )(pts_sorted, seg)
    cnt = jax.vmap(
        lambda s: jax.ops.segment_sum(jnp.ones((N,), x.dtype), s, num_segments=N)
    )(seg)

    sums_t = jnp.transpose(sums, (0, 2, 1))
    means = pl.pallas_call(
        _mean_kernel,
        grid=(B,),
        in_specs=[
            pl.BlockSpec((1, C, N), lambda b: (b, 0, 0)),
            pl.BlockSpec((1, 1, N), lambda b: (b, 0, 0)),
        ],
        out_specs=pl.BlockSpec((1, C, N), lambda b: (b, 0, 0)),
        out_shape=jax.ShapeDtypeStruct((B, C, N), x.dtype),
        compiler_params=_CompilerParams(dimension_semantics=("parallel",)),
    )(sums_t, cnt[:, None, :])

    keys = jax.random.split(key, B)
    u = jax.vmap(lambda k: jax.random.uniform(k, (L,)))(keys)
    idx = jnp.minimum(
        (u * num_vox.astype(u.dtype)[:, None]).astype(jnp.int32),
        num_vox[:, None] - 1,
    )
    return jnp.take_along_axis(means, idx[:, None, :], axis=-1)


def kernel(x):
    B, _, N = x.shape
    L1, L2 = N // 2, N // 4
    k1, k2 = jax.random.split(jax.random.key(42))
    voxel_x1 = _voxel_layer(x, _VOXEL_SIZE, L1, k1)
    voxel_x2 = _voxel_layer(voxel_x1, 2.0 * _VOXEL_SIZE, L2, k2)
    return (voxel_x1, voxel_x2)
